# baseline (speedup 1.0000x reference)
import jax
import jax.numpy as jnp
from jax import lax
from jax.experimental import pallas as pl
from jax.experimental.pallas import tpu as pltpu
from jax.experimental.pallas import tpu_sc as plsc

_H = 8
_P = 16
_R = 0.08
_G = 576
_DH = 32
_CH = 256
_TT2 = 512


def _dot(a, b, dims):
    return jax.lax.dot_general(a, b, (dims, ((), ())),
                               preferred_element_type=jnp.float32)


def _prep_body(q_ref, fm_ref, rxy_ref, wv_ref, wcat_ref, bcat_ref,
               xf_ref, yf_ref, w_ref, tbl_ref):
    q = q_ref[0]
    proj = _dot(wcat_ref[0], q, ((1,), (1,))) + bcat_ref[0]
    offx = proj[0:16, :]
    offy = proj[16:32, :]
    wl = proj[32:48, :]
    wl = wl - jnp.max(wl, axis=0, keepdims=True)
    we = jnp.exp(wl)
    w_ref[0] = we / jnp.sum(we, axis=0, keepdims=True)
    rx = rxy_ref[0, 0:1, :]
    ry = rxy_ref[0, 1:2, :]
    xf_ref[0] = (rx + _R * offx) * 23.0
    yf_ref[0] = (ry + _R * offy) * 23.0
    tbl_ref[0] = _dot(fm_ref[0], wv_ref[0], ((0,), (1,)))


def _prep(q, fm, rxy, wv, wcat, bcat, B, T, D, C):
    return pl.pallas_call(
        _prep_body,
        grid=(B, _H),
        in_specs=[
            pl.BlockSpec((1, T, D), lambda b, h: (b, 0, 0)),
            pl.BlockSpec((1, C, _G), lambda b, h: (b, 0, 0)),
            pl.BlockSpec((1, 2, T), lambda b, h: (b, 0, 0)),
            pl.BlockSpec((1, _DH, C), lambda b, h: (h, 0, 0)),
            pl.BlockSpec((1, 48, D), lambda b, h: (h, 0, 0)),
            pl.BlockSpec((1, 48, 1), lambda b, h: (h, 0, 0)),
        ],
        out_specs=[
            pl.BlockSpec((1, _P, T), lambda b, h: (b * _H + h, 0, 0)),
            pl.BlockSpec((1, _P, T), lambda b, h: (b * _H + h, 0, 0)),
            pl.BlockSpec((1, _P, T), lambda b, h: (b * _H + h, 0, 0)),
            pl.BlockSpec((1, _G, _DH), lambda b, h: (b * _H + h, 0, 0)),
        ],
        out_shape=[
            jax.ShapeDtypeStruct((B * _H, _P, T), jnp.float32),
            jax.ShapeDtypeStruct((B * _H, _P, T), jnp.float32),
            jax.ShapeDtypeStruct((B * _H, _P, T), jnp.float32),
            jax.ShapeDtypeStruct((B * _H, _G, _DH), jnp.float32),
        ],
    )(q, fm, rxy, wv, wcat, bcat)


def _sc_body(xf_hbm, yf_hbm, w_hbm, tbl_hbm, out_hbm,
             tblb, xfb, yfb, wb, outb):
    wid = lax.axis_index("s") * 2 + lax.axis_index("c")
    pltpu.sync_copy(tbl_hbm.at[wid], tblb)

    def chunk_body(i, _):
        base = i * _CH
        pltpu.sync_copy(xf_hbm.at[wid, :, pl.ds(base, _CH)], xfb)
        pltpu.sync_copy(yf_hbm.at[wid, :, pl.ds(base, _CH)], yfb)
        pltpu.sync_copy(w_hbm.at[wid, :, pl.ds(base, _CH)], wb)

        def g_body(g, _):
            def p_body(p, acc):
                xfv = xfb[p, pl.ds(g * 16, 16)]
                yfv = yfb[p, pl.ds(g * 16, 16)]
                wv = wb[p, pl.ds(g * 16, 16)]
                xt = xfv.astype(jnp.int32)
                yt = yfv.astype(jnp.int32)
                x0 = xt - jnp.where(xt.astype(jnp.float32) > xfv, 1, 0)
                y0 = yt - jnp.where(yt.astype(jnp.float32) > yfv, 1, 0)
                fx = xfv - x0.astype(jnp.float32)
                fy = yfv - y0.astype(jnp.float32)
                x1 = x0 + 1
                y1 = y0 + 1
                zero = jnp.zeros((16,), jnp.float32)
                wx0 = jnp.where((x0 >= 0) & (x0 <= 23), 1.0 - fx, zero)
                wx1 = jnp.where((x1 >= 0) & (x1 <= 23), fx, zero)
                wvfy = wv * fy
                wy0 = jnp.where((y0 >= 0) & (y0 <= 23), wv - wvfy, zero)
                wy1 = jnp.where((y1 >= 0) & (y1 <= 23), wvfy, zero)
                w00 = wx0 * wy0
                w01 = wx1 * wy0
                w10 = wx0 * wy1
                w11 = wx1 * wy1
                x0c = jnp.clip(x0, 0, 23)
                x1c = jnp.clip(x1, 0, 23)
                r0 = jnp.clip(y0, 0, 23) * 24
                r1 = jnp.clip(y1, 0, 23) * 24
                a00 = (r0 + x0c) * _DH
                a01 = (r0 + x1c) * _DH
                a10 = (r1 + x0c) * _DH
                a11 = (r1 + x1c) * _DH
                new = []
                for c in range(_DH):
                    v = (w00 * plsc.load_gather(tblb, [a00 + c])
                         + w01 * plsc.load_gather(tblb, [a01 + c])
                         + w10 * plsc.load_gather(tblb, [a10 + c])
                         + w11 * plsc.load_gather(tblb, [a11 + c]))
                    new.append(acc[c] + v)
                return tuple(new)

            acc = lax.fori_loop(
                0, _P, p_body,
                tuple(jnp.zeros((16,), jnp.float32) for _ in range(_DH)))
            for c in range(_DH):
                outb[c, pl.ds(g * 16, 16)] = acc[c]
            return 0

        lax.fori_loop(0, _CH // 16, g_body, 0)
        pltpu.sync_copy(outb, out_hbm.at[wid, :, pl.ds(base, _CH)])
        return 0

    lax.fori_loop(0, 4096 // _CH, chunk_body, 0)


def _sc_sample(xf, yf, w, tbl, BH, T):
    mesh = plsc.VectorSubcoreMesh(core_axis_name="c", subcore_axis_name="s")
    return pl.kernel(
        _sc_body,
        out_type=jax.ShapeDtypeStruct((BH, _DH, T), jnp.float32),
        mesh=mesh,
        compiler_params=pltpu.CompilerParams(needs_layout_passes=False),
        scratch_types=[
            pltpu.VMEM((_G * _DH,), jnp.float32),
            pltpu.VMEM((_P, _CH), jnp.float32),
            pltpu.VMEM((_P, _CH), jnp.float32),
            pltpu.VMEM((_P, _CH), jnp.float32),
            pltpu.VMEM((_DH, _CH), jnp.float32),
        ],
    )(xf, yf, w, tbl)


def _out_body(ctx_ref, wout_ref, bout_ref, out_ref):
    h = pl.program_id(2)
    contrib = _dot(ctx_ref[0], wout_ref[0], ((0,), (0,)))

    @pl.when(h == 0)
    def _():
        out_ref[0] = contrib + bout_ref[...]

    @pl.when(h != 0)
    def _():
        out_ref[0] += contrib


def _outproj(ctx, wout, bout, B, T, D):
    return pl.pallas_call(
        _out_body,
        grid=(B, T // _TT2, _H),
        in_specs=[
            pl.BlockSpec((1, _DH, _TT2), lambda b, t, h: (b * _H + h, 0, t)),
            pl.BlockSpec((1, _DH, D), lambda b, t, h: (h, 0, 0)),
            pl.BlockSpec((1, D), lambda b, t, h: (0, 0)),
        ],
        out_specs=pl.BlockSpec((1, _TT2, D), lambda b, t, h: (b, t, 0)),
        out_shape=jax.ShapeDtypeStruct((B, T, D), jnp.float32),
    )(ctx, wout, bout)


def kernel(q, fmap, ref_xy, W_v, W_off, b_off, W_w, b_w, W_out, b_out):
    B, T, D = q.shape
    C = fmap.shape[1]
    fm = fmap.reshape(B, C, _G)
    rxy = ref_xy.transpose(0, 2, 1)
    wv = W_v.reshape(_H, _DH, C)
    woff4 = W_off.reshape(_H, _P, 2, D)
    wcat = jnp.concatenate(
        [woff4[:, :, 0, :], woff4[:, :, 1, :], W_w.reshape(_H, _P, D)], axis=1)
    boff4 = b_off.reshape(_H, _P, 2)
    bcat = jnp.concatenate(
        [boff4[:, :, 0], boff4[:, :, 1], b_w.reshape(_H, _P)],
        axis=1).reshape(_H, 48, 1)
    wout = W_out.T.reshape(_H, _DH, D)
    bout = b_out.reshape(1, D)

    xf, yf, w, tbl = _prep(q, fm, rxy, wv, wcat, bcat, B, T, D, C)
    ctx = _sc_sample(xf, yf, w, tbl.reshape(B * _H, _G * _DH), B * _H, T)
    return _outproj(ctx, wout, bout, B, T, D)


# speedup vs baseline: 1.0589x; 1.0589x over previous
import jax
import jax.numpy as jnp
from jax import lax
from jax.experimental import pallas as pl
from jax.experimental.pallas import tpu as pltpu
from jax.experimental.pallas import tpu_sc as plsc

_H = 8
_P = 16
_R = 0.08
_G = 576
_DH = 32
_CH = 256
_TT2 = 512


def _dot(a, b, dims):
    return jax.lax.dot_general(a, b, (dims, ((), ())),
                               preferred_element_type=jnp.float32)


def _prep_body(q_ref, fm_ref, rxy_ref, wv_ref, wcat_ref, bcat_ref,
               xf_ref, yf_ref, w_ref, tbl_ref):
    q = q_ref[0]
    proj = _dot(wcat_ref[0], q, ((1,), (1,))) + bcat_ref[0]
    offx = proj[0:16, :]
    offy = proj[16:32, :]
    wl = proj[32:48, :]
    wl = wl - jnp.max(wl, axis=0, keepdims=True)
    we = jnp.exp(wl)
    w_ref[0] = we / jnp.sum(we, axis=0, keepdims=True)
    rx = rxy_ref[0, 0:1, :]
    ry = rxy_ref[0, 1:2, :]
    xf_ref[0] = (rx + _R * offx) * 23.0
    yf_ref[0] = (ry + _R * offy) * 23.0
    tbl_ref[0] = _dot(fm_ref[0], wv_ref[0], ((0,), (1,)))


def _prep(q, fm, rxy, wv, wcat, bcat, B, T, D, C):
    return pl.pallas_call(
        _prep_body,
        grid=(B, _H),
        in_specs=[
            pl.BlockSpec((1, T, D), lambda b, h: (b, 0, 0)),
            pl.BlockSpec((1, C, _G), lambda b, h: (b, 0, 0)),
            pl.BlockSpec((1, 2, T), lambda b, h: (b, 0, 0)),
            pl.BlockSpec((1, _DH, C), lambda b, h: (h, 0, 0)),
            pl.BlockSpec((1, 48, D), lambda b, h: (h, 0, 0)),
            pl.BlockSpec((1, 48, 1), lambda b, h: (h, 0, 0)),
        ],
        out_specs=[
            pl.BlockSpec((1, _P, T), lambda b, h: (b * _H + h, 0, 0)),
            pl.BlockSpec((1, _P, T), lambda b, h: (b * _H + h, 0, 0)),
            pl.BlockSpec((1, _P, T), lambda b, h: (b * _H + h, 0, 0)),
            pl.BlockSpec((1, _G, _DH), lambda b, h: (b * _H + h, 0, 0)),
        ],
        out_shape=[
            jax.ShapeDtypeStruct((B * _H, _P, T), jnp.float32),
            jax.ShapeDtypeStruct((B * _H, _P, T), jnp.float32),
            jax.ShapeDtypeStruct((B * _H, _P, T), jnp.float32),
            jax.ShapeDtypeStruct((B * _H, _G, _DH), jnp.float32),
        ],
    )(q, fm, rxy, wv, wcat, bcat)


def _sc_body(xf_hbm, yf_hbm, w_hbm, tbl_hbm, out_hbm,
             tblb, xfb, yfb, wb, outb):
    wid = lax.axis_index("s") * 2 + lax.axis_index("c")
    pltpu.sync_copy(tbl_hbm.at[wid], tblb)

    def chunk_body(i, _):
        base = i * _CH
        pltpu.sync_copy(xf_hbm.at[wid, :, pl.ds(base, _CH)], xfb)
        pltpu.sync_copy(yf_hbm.at[wid, :, pl.ds(base, _CH)], yfb)
        pltpu.sync_copy(w_hbm.at[wid, :, pl.ds(base, _CH)], wb)

        def zero_body(c, _):
            for j in range(_CH // 16):
                outb[c, pl.ds(j * 16, 16)] = jnp.zeros((16,), jnp.float32)
            return 0

        lax.fori_loop(0, _DH, zero_body, 0)

        def g_body(g, _):
            def p_body(p, _):
                sl = pl.ds(g * 16, 16)
                xfv = xfb[p, sl]
                yfv = yfb[p, sl]
                wv = wb[p, sl]
                xt = xfv.astype(jnp.int32)
                yt = yfv.astype(jnp.int32)
                x0 = xt - jnp.where(xt.astype(jnp.float32) > xfv, 1, 0)
                y0 = yt - jnp.where(yt.astype(jnp.float32) > yfv, 1, 0)
                valid = ((xfv > -1.0) & (xfv < 24.0)
                         & (yfv > -1.0) & (yfv < 24.0))
                wv = jnp.where(valid, wv, jnp.zeros((16,), jnp.float32))
                x0 = jnp.clip(x0, -1, 24)
                y0 = jnp.clip(y0, -1, 24)
                fx = xfv - x0.astype(jnp.float32)
                fy = yfv - y0.astype(jnp.float32)
                wy1 = wv * fy
                wy0 = wv - wy1
                a0 = ((y0 + 1) * 26 + (x0 + 1)) * _DH
                himask = jnp.full((16,), -65536, jnp.int32)
                for c in range(_DH):
                    r0 = plsc.load_gather(tblb, [a0 + c])
                    r1 = plsc.load_gather(tblb, [a0 + (26 * _DH + c)])
                    lo0 = plsc.bitcast(r0 << 16, jnp.float32)
                    hi0 = plsc.bitcast(r0 & himask, jnp.float32)
                    lo1 = plsc.bitcast(r1 << 16, jnp.float32)
                    hi1 = plsc.bitcast(r1 & himask, jnp.float32)
                    v0 = lo0 + fx * (hi0 - lo0)
                    v1 = lo1 + fx * (hi1 - lo1)
                    plsc.addupdate(outb.at[c, sl], wy0 * v0 + wy1 * v1)
                return 0

            lax.fori_loop(0, _P, p_body, 0)
            return 0

        lax.fori_loop(0, _CH // 16, g_body, 0)
        pltpu.sync_copy(outb, out_hbm.at[wid, :, pl.ds(base, _CH)])
        return 0

    lax.fori_loop(0, 4096 // _CH, chunk_body, 0)


def _sc_sample(xf, yf, w, tbl, BH, T):
    mesh = plsc.VectorSubcoreMesh(core_axis_name="c", subcore_axis_name="s")
    return pl.kernel(
        _sc_body,
        out_type=jax.ShapeDtypeStruct((BH, _DH, T), jnp.float32),
        mesh=mesh,
        compiler_params=pltpu.CompilerParams(needs_layout_passes=False),
        scratch_types=[
            pltpu.VMEM((27 * 26 * _DH,), jnp.int32),
            pltpu.VMEM((_P, _CH), jnp.float32),
            pltpu.VMEM((_P, _CH), jnp.float32),
            pltpu.VMEM((_P, _CH), jnp.float32),
            pltpu.VMEM((_DH, _CH), jnp.float32),
        ],
    )(xf, yf, w, tbl)


def _out_body(ctx_ref, wout_ref, bout_ref, out_ref):
    h = pl.program_id(2)
    contrib = _dot(ctx_ref[0], wout_ref[0], ((0,), (0,)))

    @pl.when(h == 0)
    def _():
        out_ref[0] = contrib + bout_ref[...]

    @pl.when(h != 0)
    def _():
        out_ref[0] += contrib


def _outproj(ctx, wout, bout, B, T, D):
    return pl.pallas_call(
        _out_body,
        grid=(B, T // _TT2, _H),
        in_specs=[
            pl.BlockSpec((1, _DH, _TT2), lambda b, t, h: (b * _H + h, 0, t)),
            pl.BlockSpec((1, _DH, D), lambda b, t, h: (h, 0, 0)),
            pl.BlockSpec((1, D), lambda b, t, h: (0, 0)),
        ],
        out_specs=pl.BlockSpec((1, _TT2, D), lambda b, t, h: (b, t, 0)),
        out_shape=jax.ShapeDtypeStruct((B, T, D), jnp.float32),
    )(ctx, wout, bout)


def kernel(q, fmap, ref_xy, W_v, W_off, b_off, W_w, b_w, W_out, b_out):
    B, T, D = q.shape
    C = fmap.shape[1]
    fm = fmap.reshape(B, C, _G)
    rxy = ref_xy.transpose(0, 2, 1)
    wv = W_v.reshape(_H, _DH, C)
    woff4 = W_off.reshape(_H, _P, 2, D)
    wcat = jnp.concatenate(
        [woff4[:, :, 0, :], woff4[:, :, 1, :], W_w.reshape(_H, _P, D)], axis=1)
    boff4 = b_off.reshape(_H, _P, 2)
    bcat = jnp.concatenate(
        [boff4[:, :, 0], boff4[:, :, 1], b_w.reshape(_H, _P)],
        axis=1).reshape(_H, 48, 1)
    wout = W_out.T.reshape(_H, _DH, D)
    bout = b_out.reshape(1, D)

    xf, yf, w, tbl = _prep(q, fm, rxy, wv, wcat, bcat, B, T, D, C)
    v4 = tbl.reshape(B * _H, 24, 24, _DH)
    vp = jnp.pad(v4, ((0, 0), (1, 2), (1, 2), (0, 0)))
    lo = lax.bitcast_convert_type(
        vp[:, :, 0:26, :].astype(jnp.bfloat16), jnp.uint16).astype(jnp.uint32)
    hi = lax.bitcast_convert_type(
        vp[:, :, 1:27, :].astype(jnp.bfloat16), jnp.uint16).astype(jnp.uint32)
    tblp = lax.bitcast_convert_type(
        lo | (hi << 16), jnp.int32).reshape(B * _H, 27 * 26 * _DH)
    ctx = _sc_sample(xf, yf, w, tblp, B * _H, T)
    return _outproj(ctx, wout, bout, B, T, D)


# speedup vs baseline: 2.2028x; 2.0803x over previous
import jax
import jax.numpy as jnp
from jax import lax
from jax.experimental import pallas as pl
from jax.experimental.pallas import tpu as pltpu
from jax.experimental.pallas import tpu_sc as plsc

_H = 8
_P = 16
_R = 0.08
_G = 576
_DH = 32
_CH = 256
_TT2 = 512


def _dot(a, b, dims):
    return jax.lax.dot_general(a, b, (dims, ((), ())),
                               preferred_element_type=jnp.float32)


def _prep_body(q_ref, fm_ref, rxy_ref, wv_ref, wcat_ref, bcat_ref,
               xf_ref, yf_ref, w_ref, tbl_ref):
    q = q_ref[0]
    proj = _dot(wcat_ref[0], q, ((1,), (1,))) + bcat_ref[0]
    offx = proj[0:16, :]
    offy = proj[16:32, :]
    wl = proj[32:48, :]
    wl = wl - jnp.max(wl, axis=0, keepdims=True)
    we = jnp.exp(wl)
    w_ref[0] = we / jnp.sum(we, axis=0, keepdims=True)
    rx = rxy_ref[0, 0:1, :]
    ry = rxy_ref[0, 1:2, :]
    xf_ref[0] = (rx + _R * offx) * 23.0
    yf_ref[0] = (ry + _R * offy) * 23.0
    tbl_ref[0] = _dot(fm_ref[0], wv_ref[0], ((0,), (1,)))


def _prep(q, fm, rxy, wv, wcat, bcat, B, T, D, C):
    return pl.pallas_call(
        _prep_body,
        grid=(B, _H),
        in_specs=[
            pl.BlockSpec((1, T, D), lambda b, h: (b, 0, 0)),
            pl.BlockSpec((1, C, _G), lambda b, h: (b, 0, 0)),
            pl.BlockSpec((1, 2, T), lambda b, h: (b, 0, 0)),
            pl.BlockSpec((1, _DH, C), lambda b, h: (h, 0, 0)),
            pl.BlockSpec((1, 48, D), lambda b, h: (h, 0, 0)),
            pl.BlockSpec((1, 48, 1), lambda b, h: (h, 0, 0)),
        ],
        out_specs=[
            pl.BlockSpec((1, _P, T), lambda b, h: (b * _H + h, 0, 0)),
            pl.BlockSpec((1, _P, T), lambda b, h: (b * _H + h, 0, 0)),
            pl.BlockSpec((1, _P, T), lambda b, h: (b * _H + h, 0, 0)),
            pl.BlockSpec((1, _G, _DH), lambda b, h: (b * _H + h, 0, 0)),
        ],
        out_shape=[
            jax.ShapeDtypeStruct((B * _H, _P, T), jnp.float32),
            jax.ShapeDtypeStruct((B * _H, _P, T), jnp.float32),
            jax.ShapeDtypeStruct((B * _H, _P, T), jnp.float32),
            jax.ShapeDtypeStruct((B * _H, _G, _DH), jnp.float32),
        ],
    )(q, fm, rxy, wv, wcat, bcat)


def _sc_body(xf_hbm, yf_hbm, w_hbm, tbl_hbm, out_hbm,
             tblb, xfb, yfb, wb, outb):
    wid = lax.axis_index("s") * 2 + lax.axis_index("c")
    pltpu.sync_copy(tbl_hbm.at[wid], tblb)

    def chunk_body(i, _):
        base = i * _CH
        pltpu.sync_copy(xf_hbm.at[wid, :, pl.ds(base, _CH)], xfb)
        pltpu.sync_copy(yf_hbm.at[wid, :, pl.ds(base, _CH)], yfb)
        pltpu.sync_copy(w_hbm.at[wid, :, pl.ds(base, _CH)], wb)

        def zero_body(c, _):
            for j in range(_CH // 16):
                outb[c, pl.ds(j * 16, 16)] = jnp.zeros((16,), jnp.float32)
            return 0

        lax.fori_loop(0, _DH, zero_body, 0)

        def g_body(g, _):
            def p_body(p, _):
                sl = pl.ds(g * 16, 16)
                xfv = xfb[p, sl]
                yfv = yfb[p, sl]
                wv = wb[p, sl]
                xt = xfv.astype(jnp.int32)
                yt = yfv.astype(jnp.int32)
                x0 = xt - jnp.where(xt.astype(jnp.float32) > xfv, 1, 0)
                y0 = yt - jnp.where(yt.astype(jnp.float32) > yfv, 1, 0)
                valid = ((xfv > -1.0) & (xfv < 24.0)
                         & (yfv > -1.0) & (yfv < 24.0))
                wv = jnp.where(valid, wv, jnp.zeros((16,), jnp.float32))
                x0 = jnp.clip(x0, -1, 24)
                y0 = jnp.clip(y0, -1, 24)
                fx = xfv - x0.astype(jnp.float32)
                fy = yfv - y0.astype(jnp.float32)
                wy1 = wv * fy
                wy0 = wv - wy1
                a0 = ((y0 + 1) * 26 + (x0 + 1)) * 33
                himask = jnp.full((16,), -65536, jnp.int32)
                for c in range(_DH):
                    r0 = plsc.load_gather(tblb, [a0 + c])
                    r1 = plsc.load_gather(tblb, [a0 + (26 * 33 + c)])
                    lo0 = plsc.bitcast(r0 << 16, jnp.float32)
                    hi0 = plsc.bitcast(r0 & himask, jnp.float32)
                    lo1 = plsc.bitcast(r1 << 16, jnp.float32)
                    hi1 = plsc.bitcast(r1 & himask, jnp.float32)
                    v0 = lo0 + fx * (hi0 - lo0)
                    v1 = lo1 + fx * (hi1 - lo1)
                    plsc.addupdate(outb.at[c, sl], wy0 * v0 + wy1 * v1)
                return 0

            lax.fori_loop(0, _P, p_body, 0)
            return 0

        lax.fori_loop(0, _CH // 16, g_body, 0)
        pltpu.sync_copy(outb, out_hbm.at[wid, :, pl.ds(base, _CH)])
        return 0

    lax.fori_loop(0, 4096 // _CH, chunk_body, 0)


def _sc_sample(xf, yf, w, tbl, BH, T):
    mesh = plsc.VectorSubcoreMesh(core_axis_name="c", subcore_axis_name="s")
    return pl.kernel(
        _sc_body,
        out_type=jax.ShapeDtypeStruct((BH, _DH, T), jnp.float32),
        mesh=mesh,
        compiler_params=pltpu.CompilerParams(needs_layout_passes=False),
        scratch_types=[
            pltpu.VMEM((27 * 26 * 33,), jnp.int32),
            pltpu.VMEM((_P, _CH), jnp.float32),
            pltpu.VMEM((_P, _CH), jnp.float32),
            pltpu.VMEM((_P, _CH), jnp.float32),
            pltpu.VMEM((_DH, _CH), jnp.float32),
        ],
    )(xf, yf, w, tbl)


def _out_body(ctx_ref, wout_ref, bout_ref, out_ref):
    h = pl.program_id(2)
    contrib = _dot(ctx_ref[0], wout_ref[0], ((0,), (0,)))

    @pl.when(h == 0)
    def _():
        out_ref[0] = contrib + bout_ref[...]

    @pl.when(h != 0)
    def _():
        out_ref[0] += contrib


def _outproj(ctx, wout, bout, B, T, D):
    return pl.pallas_call(
        _out_body,
        grid=(B, T // _TT2, _H),
        in_specs=[
            pl.BlockSpec((1, _DH, _TT2), lambda b, t, h: (b * _H + h, 0, t)),
            pl.BlockSpec((1, _DH, D), lambda b, t, h: (h, 0, 0)),
            pl.BlockSpec((1, D), lambda b, t, h: (0, 0)),
        ],
        out_specs=pl.BlockSpec((1, _TT2, D), lambda b, t, h: (b, t, 0)),
        out_shape=jax.ShapeDtypeStruct((B, T, D), jnp.float32),
    )(ctx, wout, bout)


def kernel(q, fmap, ref_xy, W_v, W_off, b_off, W_w, b_w, W_out, b_out):
    B, T, D = q.shape
    C = fmap.shape[1]
    fm = fmap.reshape(B, C, _G)
    rxy = ref_xy.transpose(0, 2, 1)
    wv = W_v.reshape(_H, _DH, C)
    woff4 = W_off.reshape(_H, _P, 2, D)
    wcat = jnp.concatenate(
        [woff4[:, :, 0, :], woff4[:, :, 1, :], W_w.reshape(_H, _P, D)], axis=1)
    boff4 = b_off.reshape(_H, _P, 2)
    bcat = jnp.concatenate(
        [boff4[:, :, 0], boff4[:, :, 1], b_w.reshape(_H, _P)],
        axis=1).reshape(_H, 48, 1)
    wout = W_out.T.reshape(_H, _DH, D)
    bout = b_out.reshape(1, D)

    xf, yf, w, tbl = _prep(q, fm, rxy, wv, wcat, bcat, B, T, D, C)
    v4 = tbl.reshape(B * _H, 24, 24, _DH)
    vp = jnp.pad(v4, ((0, 0), (1, 2), (1, 2), (0, 0)))
    lo = lax.bitcast_convert_type(
        vp[:, :, 0:26, :].astype(jnp.bfloat16), jnp.uint16).astype(jnp.uint32)
    hi = lax.bitcast_convert_type(
        vp[:, :, 1:27, :].astype(jnp.bfloat16), jnp.uint16).astype(jnp.uint32)
    packed = lax.bitcast_convert_type(lo | (hi << 16), jnp.int32)
    tblp = jnp.pad(packed, ((0, 0), (0, 0), (0, 0), (0, 1))
                   ).reshape(B * _H, 27 * 26 * 33)
    ctx = _sc_sample(xf, yf, w, tblp, B * _H, T)
    return _outproj(ctx, wout, bout, B, T, D)


# speedup vs baseline: 4.6849x; 2.1268x over previous
import jax
import jax.numpy as jnp
from jax import lax
from jax.experimental import pallas as pl
from jax.experimental.pallas import tpu as pltpu
from jax.experimental.pallas import tpu_sc as plsc

_H = 8
_P = 16
_R = 0.08
_G = 576
_DH = 32
_CH = 256
_TT2 = 512


def _dot(a, b, dims):
    return jax.lax.dot_general(a, b, (dims, ((), ())),
                               preferred_element_type=jnp.float32)


def _prep_body(q_ref, fm_ref, rxy_ref, wv_ref, wcat_ref, bcat_ref,
               xf_ref, yf_ref, w_ref, tbl_ref):
    q = q_ref[0]
    proj = _dot(wcat_ref[0], q, ((1,), (1,))) + bcat_ref[0]
    offx = proj[0:16, :]
    offy = proj[16:32, :]
    wl = proj[32:48, :]
    wl = wl - jnp.max(wl, axis=0, keepdims=True)
    we = jnp.exp(wl)
    w_ref[0] = we / jnp.sum(we, axis=0, keepdims=True)
    rx = rxy_ref[0, 0:1, :]
    ry = rxy_ref[0, 1:2, :]
    xf_ref[0] = (rx + _R * offx) * 23.0
    yf_ref[0] = (ry + _R * offy) * 23.0
    tbl_ref[0] = _dot(fm_ref[0], wv_ref[0], ((0,), (1,)))


def _prep(q, fm, rxy, wv, wcat, bcat, B, T, D, C):
    return pl.pallas_call(
        _prep_body,
        grid=(B, _H),
        in_specs=[
            pl.BlockSpec((1, T, D), lambda b, h: (b, 0, 0)),
            pl.BlockSpec((1, C, _G), lambda b, h: (b, 0, 0)),
            pl.BlockSpec((1, 2, T), lambda b, h: (b, 0, 0)),
            pl.BlockSpec((1, _DH, C), lambda b, h: (h, 0, 0)),
            pl.BlockSpec((1, 48, D), lambda b, h: (h, 0, 0)),
            pl.BlockSpec((1, 48, 1), lambda b, h: (h, 0, 0)),
        ],
        out_specs=[
            pl.BlockSpec((1, _P, T), lambda b, h: (b * _H + h, 0, 0)),
            pl.BlockSpec((1, _P, T), lambda b, h: (b * _H + h, 0, 0)),
            pl.BlockSpec((1, _P, T), lambda b, h: (b * _H + h, 0, 0)),
            pl.BlockSpec((1, _G, _DH), lambda b, h: (b * _H + h, 0, 0)),
        ],
        out_shape=[
            jax.ShapeDtypeStruct((B * _H, _P, T), jnp.float32),
            jax.ShapeDtypeStruct((B * _H, _P, T), jnp.float32),
            jax.ShapeDtypeStruct((B * _H, _P, T), jnp.float32),
            jax.ShapeDtypeStruct((B * _H, _G, _DH), jnp.float32),
        ],
    )(q, fm, rxy, wv, wcat, bcat)


def _sc_body(xf_hbm, yf_hbm, w_hbm, tbl_hbm, out_hbm,
             tblb, xfb, yfb, wb, outb):
    wid = lax.axis_index("s") * 2 + lax.axis_index("c")
    pltpu.sync_copy(tbl_hbm.at[wid], tblb)

    def chunk_body(i, _):
        base = i * _CH
        pltpu.sync_copy(xf_hbm.at[wid, :, pl.ds(base, _CH)], xfb)
        pltpu.sync_copy(yf_hbm.at[wid, :, pl.ds(base, _CH)], yfb)
        pltpu.sync_copy(w_hbm.at[wid, :, pl.ds(base, _CH)], wb)

        def g_body(g, _):
            sl = pl.ds(g * 16, 16)
            for blk in range(_DH // 8):
                zeros8 = tuple(jnp.zeros((16,), jnp.float32) for _ in range(8))

                @plsc.parallel_loop(0, _P, carry=zeros8)
                def p_body(p, acc):
                    xfv = xfb[p, sl]
                    yfv = yfb[p, sl]
                    wv = wb[p, sl]
                    xt = xfv.astype(jnp.int32)
                    yt = yfv.astype(jnp.int32)
                    x0 = xt - jnp.where(xt.astype(jnp.float32) > xfv, 1, 0)
                    y0 = yt - jnp.where(yt.astype(jnp.float32) > yfv, 1, 0)
                    valid = ((xfv > -1.0) & (xfv < 24.0)
                             & (yfv > -1.0) & (yfv < 24.0))
                    wv = jnp.where(valid, wv, jnp.zeros((16,), jnp.float32))
                    x0 = jnp.clip(x0, -1, 24)
                    y0 = jnp.clip(y0, -1, 24)
                    fx = xfv - x0.astype(jnp.float32)
                    fy = yfv - y0.astype(jnp.float32)
                    wy1 = wv * fy
                    wy0 = wv - wy1
                    a0 = ((y0 + 1) * 26 + (x0 + 1)) * 33 + (blk * 8)
                    himask = jnp.full((16,), -65536, jnp.int32)
                    new = []
                    for c in range(8):
                        r0 = plsc.load_gather(tblb, [a0 + c])
                        r1 = plsc.load_gather(tblb, [a0 + (26 * 33 + c)])
                        lo0 = plsc.bitcast(r0 << 16, jnp.float32)
                        hi0 = plsc.bitcast(r0 & himask, jnp.float32)
                        lo1 = plsc.bitcast(r1 << 16, jnp.float32)
                        hi1 = plsc.bitcast(r1 & himask, jnp.float32)
                        v0 = lo0 + fx * (hi0 - lo0)
                        v1 = lo1 + fx * (hi1 - lo1)
                        new.append(acc[c] + (wy0 * v0 + wy1 * v1))
                    return tuple(new)

                for c in range(8):
                    outb[blk * 8 + c, sl] = p_body[c]
            return 0

        lax.fori_loop(0, _CH // 16, g_body, 0)
        pltpu.sync_copy(outb, out_hbm.at[wid, :, pl.ds(base, _CH)])
        return 0

    lax.fori_loop(0, 4096 // _CH, chunk_body, 0)


def _sc_sample(xf, yf, w, tbl, BH, T):
    mesh = plsc.VectorSubcoreMesh(core_axis_name="c", subcore_axis_name="s")
    return pl.kernel(
        _sc_body,
        out_type=jax.ShapeDtypeStruct((BH, _DH, T), jnp.float32),
        mesh=mesh,
        compiler_params=pltpu.CompilerParams(needs_layout_passes=False),
        scratch_types=[
            pltpu.VMEM((27 * 26 * 33,), jnp.int32),
            pltpu.VMEM((_P, _CH), jnp.float32),
            pltpu.VMEM((_P, _CH), jnp.float32),
            pltpu.VMEM((_P, _CH), jnp.float32),
            pltpu.VMEM((_DH, _CH), jnp.float32),
        ],
    )(xf, yf, w, tbl)


def _out_body(ctx_ref, wout_ref, bout_ref, out_ref):
    h = pl.program_id(2)
    contrib = _dot(ctx_ref[0], wout_ref[0], ((0,), (0,)))

    @pl.when(h == 0)
    def _():
        out_ref[0] = contrib + bout_ref[...]

    @pl.when(h != 0)
    def _():
        out_ref[0] += contrib


def _outproj(ctx, wout, bout, B, T, D):
    return pl.pallas_call(
        _out_body,
        grid=(B, T // _TT2, _H),
        in_specs=[
            pl.BlockSpec((1, _DH, _TT2), lambda b, t, h: (b * _H + h, 0, t)),
            pl.BlockSpec((1, _DH, D), lambda b, t, h: (h, 0, 0)),
            pl.BlockSpec((1, D), lambda b, t, h: (0, 0)),
        ],
        out_specs=pl.BlockSpec((1, _TT2, D), lambda b, t, h: (b, t, 0)),
        out_shape=jax.ShapeDtypeStruct((B, T, D), jnp.float32),
    )(ctx, wout, bout)


def kernel(q, fmap, ref_xy, W_v, W_off, b_off, W_w, b_w, W_out, b_out):
    B, T, D = q.shape
    C = fmap.shape[1]
    fm = fmap.reshape(B, C, _G)
    rxy = ref_xy.transpose(0, 2, 1)
    wv = W_v.reshape(_H, _DH, C)
    woff4 = W_off.reshape(_H, _P, 2, D)
    wcat = jnp.concatenate(
        [woff4[:, :, 0, :], woff4[:, :, 1, :], W_w.reshape(_H, _P, D)], axis=1)
    boff4 = b_off.reshape(_H, _P, 2)
    bcat = jnp.concatenate(
        [boff4[:, :, 0], boff4[:, :, 1], b_w.reshape(_H, _P)],
        axis=1).reshape(_H, 48, 1)
    wout = W_out.T.reshape(_H, _DH, D)
    bout = b_out.reshape(1, D)

    xf, yf, w, tbl = _prep(q, fm, rxy, wv, wcat, bcat, B, T, D, C)
    v4 = tbl.reshape(B * _H, 24, 24, _DH)
    vp = jnp.pad(v4, ((0, 0), (1, 2), (1, 2), (0, 0)))
    lo = lax.bitcast_convert_type(
        vp[:, :, 0:26, :].astype(jnp.bfloat16), jnp.uint16).astype(jnp.uint32)
    hi = lax.bitcast_convert_type(
        vp[:, :, 1:27, :].astype(jnp.bfloat16), jnp.uint16).astype(jnp.uint32)
    packed = lax.bitcast_convert_type(lo | (hi << 16), jnp.int32)
    tblp = jnp.pad(packed, ((0, 0), (0, 0), (0, 0), (0, 1))
                   ).reshape(B * _H, 27 * 26 * 33)
    ctx = _sc_sample(xf, yf, w, tblp, B * _H, T)
    return _outproj(ctx, wout, bout, B, T, D)


# speedup vs baseline: 4.8615x; 1.0377x over previous
import jax
import jax.numpy as jnp
from jax import lax
from jax.experimental import pallas as pl
from jax.experimental.pallas import tpu as pltpu
from jax.experimental.pallas import tpu_sc as plsc

_H = 8
_P = 16
_R = 0.08
_G = 576
_DH = 32
_CH = 256
_TT2 = 512


def _dot(a, b, dims):
    return jax.lax.dot_general(a, b, (dims, ((), ())),
                               preferred_element_type=jnp.float32)


def _prep_body(q_ref, fm_ref, rxy_ref, wv_ref, wcat_ref, bcat_ref,
               xf_ref, yf_ref, w_ref, tbl_ref):
    q = q_ref[0]
    proj = _dot(wcat_ref[0], q, ((1,), (1,))) + bcat_ref[0]
    offx = proj[0:16, :]
    offy = proj[16:32, :]
    wl = proj[32:48, :]
    wl = wl - jnp.max(wl, axis=0, keepdims=True)
    we = jnp.exp(wl)
    w_ref[0] = we / jnp.sum(we, axis=0, keepdims=True)
    rx = rxy_ref[0, 0:1, :]
    ry = rxy_ref[0, 1:2, :]
    xf_ref[0] = (rx + _R * offx) * 23.0
    yf_ref[0] = (ry + _R * offy) * 23.0
    tbl_ref[0] = _dot(fm_ref[0], wv_ref[0], ((0,), (1,)))


def _prep(q, fm, rxy, wv, wcat, bcat, B, T, D, C):
    return pl.pallas_call(
        _prep_body,
        grid=(B, _H),
        in_specs=[
            pl.BlockSpec((1, T, D), lambda b, h: (b, 0, 0)),
            pl.BlockSpec((1, C, _G), lambda b, h: (b, 0, 0)),
            pl.BlockSpec((1, 2, T), lambda b, h: (b, 0, 0)),
            pl.BlockSpec((1, _DH, C), lambda b, h: (h, 0, 0)),
            pl.BlockSpec((1, 48, D), lambda b, h: (h, 0, 0)),
            pl.BlockSpec((1, 48, 1), lambda b, h: (h, 0, 0)),
        ],
        out_specs=[
            pl.BlockSpec((1, _P, T), lambda b, h: (b * _H + h, 0, 0)),
            pl.BlockSpec((1, _P, T), lambda b, h: (b * _H + h, 0, 0)),
            pl.BlockSpec((1, _P, T), lambda b, h: (b * _H + h, 0, 0)),
            pl.BlockSpec((1, _G, _DH), lambda b, h: (b * _H + h, 0, 0)),
        ],
        out_shape=[
            jax.ShapeDtypeStruct((B * _H, _P, T), jnp.float32),
            jax.ShapeDtypeStruct((B * _H, _P, T), jnp.float32),
            jax.ShapeDtypeStruct((B * _H, _P, T), jnp.float32),
            jax.ShapeDtypeStruct((B * _H, _G, _DH), jnp.float32),
        ],
    )(q, fm, rxy, wv, wcat, bcat)


def _sc_body(xf_hbm, yf_hbm, w_hbm, tbl_hbm, out_hbm,
             tblb, xfb, yfb, wb, outb):
    wid = lax.axis_index("s") * 2 + lax.axis_index("c")
    pltpu.sync_copy(tbl_hbm.at[wid], tblb)

    def chunk_body(i, _):
        base = i * _CH
        pltpu.sync_copy(xf_hbm.at[wid, :, pl.ds(base, _CH)], xfb)
        pltpu.sync_copy(yf_hbm.at[wid, :, pl.ds(base, _CH)], yfb)
        pltpu.sync_copy(w_hbm.at[wid, :, pl.ds(base, _CH)], wb)

        def g_body(g, _):
            sl = pl.ds(g * 16, 16)
            for blk in range(_DH // 16):
                zeros16 = tuple(jnp.zeros((16,), jnp.float32) for _ in range(16))

                @plsc.parallel_loop(0, _P, carry=zeros16)
                def p_body(p, acc):
                    xfv = xfb[p, sl]
                    yfv = yfb[p, sl]
                    wv = wb[p, sl]
                    xt = xfv.astype(jnp.int32)
                    yt = yfv.astype(jnp.int32)
                    x0 = xt - jnp.where(xt.astype(jnp.float32) > xfv, 1, 0)
                    y0 = yt - jnp.where(yt.astype(jnp.float32) > yfv, 1, 0)
                    valid = ((xfv > -1.0) & (xfv < 24.0)
                             & (yfv > -1.0) & (yfv < 24.0))
                    wv = jnp.where(valid, wv, jnp.zeros((16,), jnp.float32))
                    x0 = jnp.clip(x0, -1, 24)
                    y0 = jnp.clip(y0, -1, 24)
                    fx = xfv - x0.astype(jnp.float32)
                    fy = yfv - y0.astype(jnp.float32)
                    wy1 = wv * fy
                    wy0 = wv - wy1
                    a0 = ((y0 + 1) * 26 + (x0 + 1)) * 33 + (blk * 16)
                    himask = jnp.full((16,), -65536, jnp.int32)
                    new = []
                    for c in range(16):
                        r0 = plsc.load_gather(tblb, [a0 + c])
                        r1 = plsc.load_gather(tblb, [a0 + (26 * 33 + c)])
                        lo0 = plsc.bitcast(r0 << 16, jnp.float32)
                        hi0 = plsc.bitcast(r0 & himask, jnp.float32)
                        lo1 = plsc.bitcast(r1 << 16, jnp.float32)
                        hi1 = plsc.bitcast(r1 & himask, jnp.float32)
                        v0 = lo0 + fx * (hi0 - lo0)
                        v1 = lo1 + fx * (hi1 - lo1)
                        new.append(acc[c] + (wy0 * v0 + wy1 * v1))
                    return tuple(new)

                for c in range(16):
                    outb[blk * 16 + c, sl] = p_body[c]
            return 0

        lax.fori_loop(0, _CH // 16, g_body, 0)
        pltpu.sync_copy(outb, out_hbm.at[wid, :, pl.ds(base, _CH)])
        return 0

    lax.fori_loop(0, 4096 // _CH, chunk_body, 0)


def _sc_sample(xf, yf, w, tbl, BH, T):
    mesh = plsc.VectorSubcoreMesh(core_axis_name="c", subcore_axis_name="s")
    return pl.kernel(
        _sc_body,
        out_type=jax.ShapeDtypeStruct((BH, _DH, T), jnp.float32),
        mesh=mesh,
        compiler_params=pltpu.CompilerParams(needs_layout_passes=False),
        scratch_types=[
            pltpu.VMEM((27 * 26 * 33,), jnp.int32),
            pltpu.VMEM((_P, _CH), jnp.float32),
            pltpu.VMEM((_P, _CH), jnp.float32),
            pltpu.VMEM((_P, _CH), jnp.float32),
            pltpu.VMEM((_DH, _CH), jnp.float32),
        ],
    )(xf, yf, w, tbl)


def _out_body(ctx_ref, wout_ref, bout_ref, out_ref):
    h = pl.program_id(2)
    contrib = _dot(ctx_ref[0], wout_ref[0], ((0,), (0,)))

    @pl.when(h == 0)
    def _():
        out_ref[0] = contrib + bout_ref[...]

    @pl.when(h != 0)
    def _():
        out_ref[0] += contrib


def _outproj(ctx, wout, bout, B, T, D):
    return pl.pallas_call(
        _out_body,
        grid=(B, T // _TT2, _H),
        in_specs=[
            pl.BlockSpec((1, _DH, _TT2), lambda b, t, h: (b * _H + h, 0, t)),
            pl.BlockSpec((1, _DH, D), lambda b, t, h: (h, 0, 0)),
            pl.BlockSpec((1, D), lambda b, t, h: (0, 0)),
        ],
        out_specs=pl.BlockSpec((1, _TT2, D), lambda b, t, h: (b, t, 0)),
        out_shape=jax.ShapeDtypeStruct((B, T, D), jnp.float32),
    )(ctx, wout, bout)


def kernel(q, fmap, ref_xy, W_v, W_off, b_off, W_w, b_w, W_out, b_out):
    B, T, D = q.shape
    C = fmap.shape[1]
    fm = fmap.reshape(B, C, _G)
    rxy = ref_xy.transpose(0, 2, 1)
    wv = W_v.reshape(_H, _DH, C)
    woff4 = W_off.reshape(_H, _P, 2, D)
    wcat = jnp.concatenate(
        [woff4[:, :, 0, :], woff4[:, :, 1, :], W_w.reshape(_H, _P, D)], axis=1)
    boff4 = b_off.reshape(_H, _P, 2)
    bcat = jnp.concatenate(
        [boff4[:, :, 0], boff4[:, :, 1], b_w.reshape(_H, _P)],
        axis=1).reshape(_H, 48, 1)
    wout = W_out.T.reshape(_H, _DH, D)
    bout = b_out.reshape(1, D)

    xf, yf, w, tbl = _prep(q, fm, rxy, wv, wcat, bcat, B, T, D, C)
    v4 = tbl.reshape(B * _H, 24, 24, _DH)
    vp = jnp.pad(v4, ((0, 0), (1, 2), (1, 2), (0, 0)))
    lo = lax.bitcast_convert_type(
        vp[:, :, 0:26, :].astype(jnp.bfloat16), jnp.uint16).astype(jnp.uint32)
    hi = lax.bitcast_convert_type(
        vp[:, :, 1:27, :].astype(jnp.bfloat16), jnp.uint16).astype(jnp.uint32)
    packed = lax.bitcast_convert_type(lo | (hi << 16), jnp.int32)
    tblp = jnp.pad(packed, ((0, 0), (0, 0), (0, 0), (0, 1))
                   ).reshape(B * _H, 27 * 26 * 33)
    ctx = _sc_sample(xf, yf, w, tblp, B * _H, T)
    return _outproj(ctx, wout, bout, B, T, D)
